# baseline (speedup 1.0000x reference)
import functools

import jax
import jax.numpy as jnp
from jax import lax
from jax.experimental import pallas as pl
from jax.experimental.pallas import tpu as pltpu
from jax.experimental.pallas import tpu_sc as plsc

B0, H, D = 4096, 50, 64
V = 1001
NC, NS = 2, 16
NW = NC * NS
BT = 128
DT = 8
NDT = D // DT
HC = 2
NCH = H // HC
L = 16
GP = D + 1
TS = 128
PIECES = (0, 128, 256, 384, 512, 640, 768, V - TS)

_MESH = plsc.VectorSubcoreMesh(core_axis_name="c", subcore_axis_name="s")


@functools.partial(
    pl.kernel,
    out_type=jax.ShapeDtypeStruct((H, NDT, NW, DT, BT), jnp.float32),
    mesh=_MESH,
    scratch_types=[
        pltpu.VMEM((BT, H), jnp.int32),
        pltpu.VMEM((TS, D), jnp.float32),
        pltpu.VMEM((V * GP,), jnp.float32),
        pltpu.VMEM((HC, NDT, DT, BT + 1), jnp.float32),
        pltpu.VMEM((HC, NDT, DT, BT + 1), jnp.float32),
        pltpu.SemaphoreType.DMA,
        pltpu.SemaphoreType.DMA,
    ],
    compiler_params=pltpu.CompilerParams(
        use_tc_tiling_on_sc=False, needs_layout_passes=False),
)
def _gather_rows(ids_hbm, table_hbm, out_hbm, idx_v, tstage, tabp,
                 t0, t1, ss0, ss1):
    wid = lax.axis_index("s") * NC + lax.axis_index("c")
    tbuf = (t0, t1)
    ssem = (ss0, ss1)
    iota = lax.iota(jnp.int32, L)

    pltpu.sync_copy(ids_hbm.at[pl.ds(wid * BT, BT)], idx_v)

    for base in PIECES:
        pltpu.sync_copy(table_hbm.at[pl.ds(base, TS)], tstage)

        def stage_b8(b8, carry, base=base):
            b0 = b8 * 8
            for bi in range(8):
                b = b0 + bi
                vs = [tstage[b, pl.ds(L * j, L)] for j in range(D // L)]
                for j in range(D // L):
                    plsc.store_scatter(
                        tabp, [iota + ((base + b) * GP + L * j)], vs[j])
            return carry

        lax.fori_loop(0, TS // 8, stage_b8, 0)

    def start_store(c):
        return pltpu.async_copy(
            tbuf[c % 2].at[:, :, :, pl.ds(0, BT)],
            out_hbm.at[pl.ds(c * HC, HC), :, wid], ssem[c % 2])

    def chunk(c):
        t = tbuf[c % 2]
        idx65 = [[plsc.load_gather(
                      idx_v, [iota + (L * k), jnp.full((L,), c * HC + h,
                                                       jnp.int32)]) * GP
                  for k in range(BT // L)] for h in range(HC)]

        def per_d(d, carry):
            dt = lax.shift_right_logical(d, 3)
            di = lax.bitwise_and(d, 7)
            vs = []
            for h in range(HC):
                for k in range(BT // L):
                    vs.append(plsc.load_gather(tabp, [idx65[h][k] + d]))
            i = 0
            for h in range(HC):
                for k in range(BT // L):
                    t[h, dt, di, pl.ds(L * k, L)] = vs[i]
                    i += 1
            return carry

        lax.fori_loop(0, D, per_d, 0, unroll=2)

    stores = [None] * NCH
    for c in range(NCH):
        if c >= 2:
            stores[c - 2].wait()
        chunk(c)
        stores[c] = start_store(c)
    stores[NCH - 2].wait()
    stores[NCH - 1].wait()


def kernel(ids, table):
    out5 = _gather_rows(ids, table)
    return jnp.transpose(out5, (2, 4, 0, 1, 3)).reshape(B0, H, D)


# speedup vs baseline: 1.0205x; 1.0205x over previous
import functools

import jax
import jax.numpy as jnp
from jax import lax
from jax.experimental import pallas as pl
from jax.experimental.pallas import tpu as pltpu
from jax.experimental.pallas import tpu_sc as plsc

B0, H, D = 4096, 50, 64
V = 1001
NC, NS = 2, 16
NW = NC * NS
BT = 128
DT = 8
NDT = D // DT
HC = 2
NCH = H // HC
L = 16
GP = D + 1
TS = 128
PIECES = (0, 128, 256, 384, 512, 640, 768, V - TS)

_MESH = plsc.VectorSubcoreMesh(core_axis_name="c", subcore_axis_name="s")


@functools.partial(
    pl.kernel,
    out_type=jax.ShapeDtypeStruct((H, NDT, NW, DT, BT), jnp.float32),
    mesh=_MESH,
    scratch_types=[
        pltpu.VMEM((BT, H), jnp.int32),
        pltpu.VMEM((TS, D), jnp.float32),
        pltpu.VMEM((V * GP,), jnp.float32),
        pltpu.VMEM((HC, NDT, DT, BT + 1), jnp.float32),
        pltpu.VMEM((HC, NDT, DT, BT + 1), jnp.float32),
        pltpu.SemaphoreType.DMA,
        pltpu.SemaphoreType.DMA,
    ],
    compiler_params=pltpu.CompilerParams(
        use_tc_tiling_on_sc=False, needs_layout_passes=False),
)
def _gather_rows(ids_hbm, table_hbm, out_hbm, idx_v, tstage, tabp,
                 t0, t1, ss0, ss1):
    wid = lax.axis_index("s") * NC + lax.axis_index("c")
    tbuf = (t0, t1)
    ssem = (ss0, ss1)
    iota = lax.iota(jnp.int32, L)

    pltpu.sync_copy(ids_hbm.at[pl.ds(wid * BT, BT)], idx_v)

    for base in PIECES:
        pltpu.sync_copy(table_hbm.at[pl.ds(base, TS)], tstage)

        def stage_b8(b8, carry, base=base):
            b0 = b8 * 8
            for bi in range(8):
                b = b0 + bi
                vs = [tstage[b, pl.ds(L * j, L)] for j in range(D // L)]
                for j in range(D // L):
                    plsc.store_scatter(
                        tabp, [iota + ((base + b) * GP + L * j)], vs[j])
            return carry

        lax.fori_loop(0, TS // 8, stage_b8, 0)

    def start_store(c):
        return pltpu.async_copy(
            tbuf[c % 2].at[:, :, :, pl.ds(0, BT)],
            out_hbm.at[pl.ds(c * HC, HC), :, wid], ssem[c % 2])

    def chunk(c):
        t = tbuf[c % 2]
        idx65 = [[plsc.load_gather(
                      idx_v, [iota + (L * k), jnp.full((L,), c * HC + h,
                                                       jnp.int32)]) * GP
                  for k in range(BT // L)] for h in range(HC)]

        def per_d(d, carry):
            dt = lax.shift_right_logical(d, 3)
            di = lax.bitwise_and(d, 7)
            vs = []
            for h in range(HC):
                for k in range(BT // L):
                    vs.append(plsc.load_gather(tabp, [idx65[h][k] + d]))
            i = 0
            for h in range(HC):
                for k in range(BT // L):
                    t[h, dt, di, pl.ds(L * k, L)] = vs[i]
                    i += 1
            return carry

        lax.fori_loop(0, D, per_d, 0)

    stores = [None] * NCH
    for c in range(NCH):
        if c >= 2:
            stores[c - 2].wait()
        chunk(c)
        stores[c] = start_store(c)
    stores[NCH - 2].wait()
    stores[NCH - 1].wait()


def kernel(ids, table):
    out5 = _gather_rows(ids, table)
    return jnp.transpose(out5, (2, 4, 0, 1, 3)).reshape(B0, H, D)


# speedup vs baseline: 1.1898x; 1.1659x over previous
import functools

import jax
import jax.numpy as jnp
from jax import lax
from jax.experimental import pallas as pl
from jax.experimental.pallas import tpu as pltpu
from jax.experimental.pallas import tpu_sc as plsc

B0, H, D = 4096, 50, 64
V = 1001
NC, NS = 2, 16
NW = NC * NS
BT = 128
DT = 8
NDT = D // DT
HC = 2
NCH = H // HC
L = 16
GP = D + 1
TS = 128
PIECES = (0, 128, 256, 384, 512, 640, 768, V - TS)

_MESH = plsc.VectorSubcoreMesh(core_axis_name="c", subcore_axis_name="s")


@functools.partial(
    pl.kernel,
    out_type=jax.ShapeDtypeStruct((H, NDT, NW, DT, BT), jnp.float32),
    mesh=_MESH,
    scratch_types=[
        pltpu.VMEM((BT, H), jnp.int32),
        pltpu.VMEM((TS, D), jnp.float32),
        pltpu.VMEM((V * GP,), jnp.float32),
        pltpu.VMEM((HC, NDT, DT, BT), jnp.float32),
        pltpu.VMEM((HC, NDT, DT, BT), jnp.float32),
        pltpu.SemaphoreType.DMA,
        pltpu.SemaphoreType.DMA,
    ],
    compiler_params=pltpu.CompilerParams(
        use_tc_tiling_on_sc=False, needs_layout_passes=False),
)
def _gather_rows(ids_hbm, table_hbm, out_hbm, idx_v, tstage, tabp,
                 t0, t1, ss0, ss1):
    wid = lax.axis_index("s") * NC + lax.axis_index("c")
    tbuf = (t0, t1)
    ssem = (ss0, ss1)
    iota = lax.iota(jnp.int32, L)

    pltpu.sync_copy(ids_hbm.at[pl.ds(wid * BT, BT)], idx_v)

    for base in PIECES:
        pltpu.sync_copy(table_hbm.at[pl.ds(base, TS)], tstage)

        def stage_b8(b8, carry, base=base):
            b0 = b8 * 8
            for bi in range(8):
                b = b0 + bi
                vs = [tstage[b, pl.ds(L * j, L)] for j in range(D // L)]
                for j in range(D // L):
                    plsc.store_scatter(
                        tabp, [iota + ((base + b) * GP + L * j)], vs[j])
            return carry

        lax.fori_loop(0, TS // 8, stage_b8, 0)

    def start_store(c):
        return pltpu.async_copy(
            tbuf[c % 2], out_hbm.at[pl.ds(c * HC, HC), :, wid], ssem[c % 2])

    def chunk(c):
        t = tbuf[c % 2]
        idx65 = [[plsc.load_gather(
                      idx_v, [iota + (L * k), jnp.full((L,), c * HC + h,
                                                       jnp.int32)]) * GP
                  for k in range(BT // L)] for h in range(HC)]

        def per_d(d, carry):
            dt = lax.shift_right_logical(d, 3)
            di = lax.bitwise_and(d, 7)
            vs = []
            for h in range(HC):
                for k in range(BT // L):
                    vs.append(plsc.load_gather(tabp, [idx65[h][k] + d]))
            i = 0
            for h in range(HC):
                for k in range(BT // L):
                    t[h, dt, di, pl.ds(L * k, L)] = vs[i]
                    i += 1
            return carry

        lax.fori_loop(0, D, per_d, 0)

    stores = [None] * NCH
    for c in range(NCH):
        if c >= 2:
            stores[c - 2].wait()
        chunk(c)
        stores[c] = start_store(c)
    stores[NCH - 2].wait()
    stores[NCH - 1].wait()


def kernel(ids, table):
    out5 = _gather_rows(ids, table)
    return jnp.transpose(out5, (2, 4, 0, 1, 3)).reshape(B0, H, D)


# speedup vs baseline: 1.2697x; 1.0672x over previous
import functools

import jax
import jax.numpy as jnp
from jax import lax
from jax.experimental import pallas as pl
from jax.experimental.pallas import tpu as pltpu
from jax.experimental.pallas import tpu_sc as plsc

B0, H, D = 4096, 50, 64
V = 1001
NC, NS = 2, 16
NW = NC * NS
BT = 128
DT = 8
NDT = D // DT
HC = 2
NCH = H // HC
L = 16
GP = D + 1
TS = 128
PIECES = (0, 128, 256, 384, 512, 640, 768, V - TS)

_MESH = plsc.VectorSubcoreMesh(core_axis_name="c", subcore_axis_name="s")


@functools.partial(
    pl.kernel,
    out_type=jax.ShapeDtypeStruct((H, NDT, NW, DT, BT), jnp.float32),
    mesh=_MESH,
    scratch_types=[
        pltpu.VMEM((BT, H), jnp.int32),
        pltpu.VMEM((2 * TS, D), jnp.float32),
        pltpu.VMEM((V * GP,), jnp.float32),
        pltpu.VMEM((HC, NDT, DT, BT), jnp.float32),
        pltpu.VMEM((HC, NDT, DT, BT), jnp.float32),
        pltpu.SemaphoreType.DMA,
        pltpu.SemaphoreType.DMA,
    ],
    compiler_params=pltpu.CompilerParams(
        use_tc_tiling_on_sc=False, needs_layout_passes=False),
)
def _gather_rows(ids_hbm, table_hbm, out_hbm, idx_v, tstage, tabp,
                 t0, t1, ss0, ss1):
    wid = lax.axis_index("s") * NC + lax.axis_index("c")
    tbuf = (t0, t1)
    ssem = (ss0, ss1)
    iota = lax.iota(jnp.int32, L)

    pltpu.sync_copy(ids_hbm.at[pl.ds(wid * BT, BT)], idx_v)

    cps = [None] * len(PIECES)
    cps[0] = pltpu.async_copy(
        table_hbm.at[pl.ds(PIECES[0], TS)], tstage.at[pl.ds(0, TS)], ss0)
    for p, base in enumerate(PIECES):
        half = (p % 2) * TS
        if p + 1 < len(PIECES):
            nhalf = ((p + 1) % 2) * TS
            cps[p + 1] = pltpu.async_copy(
                table_hbm.at[pl.ds(PIECES[p + 1], TS)],
                tstage.at[pl.ds(nhalf, TS)], ss1 if (p + 1) % 2 else ss0)
        cps[p].wait()

        def stage_b8(b8, carry, base=base, half=half):
            b0 = b8 * 8
            for bi in range(8):
                b = b0 + bi
                vs = [tstage[half + b, pl.ds(L * j, L)] for j in range(D // L)]
                for j in range(D // L):
                    plsc.store_scatter(
                        tabp, [iota + ((base + b) * GP + L * j)], vs[j])
            return carry

        lax.fori_loop(0, TS // 8, stage_b8, 0)

    def start_store(c):
        return pltpu.async_copy(
            tbuf[c % 2], out_hbm.at[pl.ds(c * HC, HC), :, wid], ssem[c % 2])

    def chunk(c):
        t = tbuf[c % 2]
        idx65 = [[plsc.load_gather(
                      idx_v, [iota + (L * k), jnp.full((L,), c * HC + h,
                                                       jnp.int32)]) * GP
                  for k in range(BT // L)] for h in range(HC)]

        def per_d(d, carry):
            dt = lax.shift_right_logical(d, 3)
            di = lax.bitwise_and(d, 7)
            vs = []
            for h in range(HC):
                for k in range(BT // L):
                    vs.append(plsc.load_gather(tabp, [idx65[h][k] + d]))
            i = 0
            for h in range(HC):
                for k in range(BT // L):
                    t[h, dt, di, pl.ds(L * k, L)] = vs[i]
                    i += 1
            return carry

        lax.fori_loop(0, D, per_d, 0)

    stores = [None] * NCH
    for c in range(NCH):
        if c >= 2:
            stores[c - 2].wait()
        chunk(c)
        stores[c] = start_store(c)
    stores[NCH - 2].wait()
    stores[NCH - 1].wait()


def kernel(ids, table):
    out5 = _gather_rows(ids, table)
    return jnp.transpose(out5, (2, 4, 0, 1, 3)).reshape(B0, H, D)
